# Initial kernel scaffold; baseline (speedup 1.0000x reference)
import jax
import jax.numpy as jnp
from jax.experimental import pallas as pl


def kernel(inputs, token_table, pos_table):
    raise NotImplementedError("write your pallas kernel here")


# speedup vs baseline: 3.9705x; 3.9705x over previous
import functools

import jax
import jax.numpy as jnp
from jax import lax
from jax.experimental import pallas as pl
from jax.experimental.pallas import tpu as pltpu
from jax.experimental.pallas import tpu_sc as plsc


def _embed_kernel(S, F, D, SUB, GPC, num_chunks, tail, NC, NW,
                  pos_base, pos_off, pos_rows,
                  idx_hbm, tok_hbm, pos_hbm, out_hbm,
                  idx_v, rows_v, pos_v, sem):
    C = SUB * GPC
    w = lax.axis_index("s") * NC + lax.axis_index("c")

    pltpu.sync_copy(pos_hbm.at[pl.ds(pos_base, pos_rows), :], pos_v)
    onehot = lax.broadcasted_iota(jnp.int32, (D,), 0) == (D - 1)
    pvecs = []
    for f in range(F):
        v = pos_v[pos_off + f, :]
        scal = jnp.sum(jnp.where(onehot, v, jnp.zeros_like(v)))
        pvecs.append(jnp.full((D,), scal, dtype=jnp.float32))

    n_mine = (num_chunks - 1 - w) // NW + 1

    def chunk_body(k, carry):
        c = w + k * NW
        pltpu.sync_copy(idx_hbm.at[pl.ds(c * GPC, GPC), :], idx_v)
        copies = [
            pltpu.async_copy(tok_hbm.at[idx_v.at[j]],
                             rows_v.at[pl.ds(j * SUB, SUB)], sem)
            for j in range(GPC)
        ]
        for cp in copies:
            cp.wait()

        def add_body(g, acc):
            r0 = g * F
            for j in range(F):
                rows_v[r0 + j, :] = rows_v[r0 + j, :] + pvecs[j]
            return acc

        lax.fori_loop(0, C // F, add_body, 0)

        base = c * C
        if tail == C:
            pltpu.sync_copy(rows_v, out_hbm.at[pl.ds(base, C)])
        else:
            @pl.when(c != num_chunks - 1)
            def _():
                pltpu.sync_copy(rows_v, out_hbm.at[pl.ds(base, C)])

            @pl.when(c == num_chunks - 1)
            def _():
                pltpu.sync_copy(rows_v.at[pl.ds(0, tail)],
                                out_hbm.at[pl.ds(base, tail)])
        return carry

    lax.fori_loop(0, n_mine, chunk_body, 0)


def kernel(inputs, token_table, pos_table):
    S, F = inputs.shape
    V, D = token_table.shape
    N = S * F
    SUB = 120
    GPC = 16
    C = SUB * GPC
    assert C % F == 0 and SUB % 8 == 0
    num_chunks = -(-N // C)
    n_pad = num_chunks * C
    tail = N - (num_chunks - 1) * C

    info = plsc.get_sparse_core_info()
    NC, NS = info.num_cores, info.num_subcores
    NW = NC * NS

    pos_base = (S // 8) * 8
    pos_rows = -(-(S - pos_base + F) // 8) * 8
    if pos_base + pos_rows > V:
        pos_base = ((V - pos_rows) // 8) * 8
    pos_off = S - pos_base

    idx = inputs.reshape(-1)
    if n_pad != N:
        idx = jnp.pad(idx, (0, n_pad - N))
    idx2 = idx.reshape(n_pad // SUB, SUB)

    body = functools.partial(_embed_kernel, S, F, D, SUB, GPC, num_chunks,
                             tail, NC, NW, pos_base, pos_off, pos_rows)
    mesh = plsc.VectorSubcoreMesh(core_axis_name="c", subcore_axis_name="s")
    out = pl.kernel(
        body,
        mesh=mesh,
        compiler_params=pltpu.CompilerParams(use_tc_tiling_on_sc=False,
                                             needs_layout_passes=False),
        out_type=jax.ShapeDtypeStruct((N, D), jnp.float32),
        scratch_types=[
            pltpu.VMEM((GPC, SUB), jnp.int32),
            pltpu.VMEM((C, D), jnp.float32),
            pltpu.VMEM((pos_rows, D), jnp.float32),
            pltpu.SemaphoreType.DMA,
        ],
    )(idx2, token_table, pos_table)
    return out.reshape(S, F, D)


# speedup vs baseline: 6.3227x; 1.5924x over previous
import functools

import jax
import jax.numpy as jnp
from jax import lax
from jax.experimental import pallas as pl
from jax.experimental.pallas import tpu as pltpu
from jax.experimental.pallas import tpu_sc as plsc


def _gather_kernel(F, CS, num_chunks, NC, NW,
                   idx_hbm, tok_hbm, inter_hbm, idx_v, rows_v, sem):
    w = lax.axis_index("s") * NC + lax.axis_index("c")
    C = F * CS
    n_mine = (num_chunks - 1 - w) // NW + 1

    def chunk_body(k, carry):
        sb = w + k * NW
        pltpu.sync_copy(idx_hbm.at[:, pl.ds(sb * CS, CS)], idx_v)
        copies = [
            pltpu.async_copy(tok_hbm.at[idx_v.at[f]],
                             rows_v.at[pl.ds(f * CS, CS)], sem)
            for f in range(F)
        ]
        for cp in copies:
            cp.wait()
        pltpu.sync_copy(rows_v, inter_hbm.at[pl.ds(sb * C, C)])
        return carry

    lax.fori_loop(0, n_mine, chunk_body, 0)


def _addpos_kernel(F, D, CS, num_chunks, NC, NW,
                   in_hbm, pos_hbm, out_hbm, rows_v, buf_v, pos_v):
    w = lax.axis_index("s") * NC + lax.axis_index("c")
    R = F * CS * D // 128
    iota_v = lax.broadcasted_iota(jnp.int32, (D,), 0)
    row_c = lax.shift_right_logical(iota_v, 3)
    lane_c = (iota_v & 7) * D
    pltpu.sync_copy(pos_hbm, pos_v)
    n_mine = (num_chunks - 1 - w) // NW + 1

    def chunk_body(k, carry):
        sb = w + k * NW
        pltpu.sync_copy(in_hbm.at[pl.ds(sb * R, R)], rows_v)

        def f_body(f, c2):
            pv = pos_v[f >> 3, pl.ds((f & 7) * D, D)]

            def d_body(d, c3):
                lane_v = lane_c + d
                for kk in range(CS // D):
                    row_v = row_c + (f * D + 2 * kk)
                    vals = plsc.load_gather(rows_v, [row_v, lane_v])
                    buf_v[f * D + d, pl.ds(kk * D, D)] = vals + pv
                return c3

            lax.fori_loop(0, D, d_body, 0)
            return c2

        lax.fori_loop(0, F, f_body, 0)
        pltpu.sync_copy(buf_v, out_hbm.at[:, pl.ds(sb * CS, CS)])
        return carry

    lax.fori_loop(0, n_mine, chunk_body, 0)


def kernel(inputs, token_table, pos_table):
    S, F = inputs.shape
    V, D = token_table.shape
    assert D == 16 and F <= 16
    CS = 128
    num_chunks = S // CS
    s_main = num_chunks * CS
    C = F * CS

    info = plsc.get_sparse_core_info()
    NC, NS = info.num_cores, info.num_subcores
    NW = NC * NS

    pos_block = pos_table[S:, F:]
    pos_flat = jnp.broadcast_to(pos_block, (F, D)).reshape(-1)
    pos2 = jnp.pad(pos_flat, (0, 256 - F * D)).reshape(2, 128)

    idx_t = inputs.T

    mesh = plsc.VectorSubcoreMesh(core_axis_name="c", subcore_axis_name="s")

    gather_body = functools.partial(_gather_kernel, F, CS, num_chunks, NC, NW)
    inter = pl.kernel(
        gather_body,
        mesh=mesh,
        compiler_params=pltpu.CompilerParams(use_tc_tiling_on_sc=False,
                                             needs_layout_passes=False),
        out_type=jax.ShapeDtypeStruct((num_chunks * C, D), jnp.float32),
        scratch_types=[
            pltpu.VMEM((F, CS), jnp.int32),
            pltpu.VMEM((C, D), jnp.float32),
            pltpu.SemaphoreType.DMA,
        ],
    )(idx_t, token_table)

    in128 = inter.reshape(num_chunks * C * D // 128, 128)

    addpos_body = functools.partial(_addpos_kernel, F, D, CS, num_chunks,
                                    NC, NW)
    out_t = pl.kernel(
        addpos_body,
        mesh=mesh,
        compiler_params=pltpu.CompilerParams(needs_layout_passes=False),
        out_type=jax.ShapeDtypeStruct((F * D, S), jnp.float32),
        scratch_types=[
            pltpu.VMEM((C * D // 128, 128), jnp.float32),
            pltpu.VMEM((F * D, CS), jnp.float32),
            pltpu.VMEM((2, 128), jnp.float32),
        ],
    )(in128, pos2)

    out = out_t.reshape(F, D, S).transpose(2, 0, 1)
    if s_main < S:
        tail = jnp.take(token_table, inputs[s_main:], axis=0)
        tail = tail + jnp.broadcast_to(pos_block, (F, D))[None]
        out = lax.dynamic_update_slice(out, tail, (s_main, 0, 0))
    return out
